# baseline (speedup 1.0000x reference)
import jax
import jax.numpy as jnp
from jax import lax
from jax.experimental import pallas as pl
from jax.experimental.pallas import tpu as pltpu
from jax.experimental.pallas import tpu_sc as plsc

_BATCH = 4
_SEQ = 4096
_HID = 4096
_LANES = 16
_HSEQ = _SEQ // 2
_HHID = _HID // 2
_NVREG_H = _HSEQ // _LANES


def _sc_body(ids_hbm, hs_hbm, out_hbm, ids_v, row_v, s0, s1, s2, s3, s4, s5, s6, s7):
    b = lax.axis_index("s")

    @pl.when(b < _BATCH)
    def _():
        c0 = pltpu.make_async_copy(
            ids_hbm.at[b, pl.ds(0, _HSEQ)], ids_v.at[pl.ds(0, _HSEQ)], s0
        )
        c0.start()
        c1 = pltpu.make_async_copy(
            ids_hbm.at[b, pl.ds(_HSEQ, _HSEQ)],
            ids_v.at[pl.ds(_HSEQ, _HSEQ)],
            s1,
        )
        c1.start()
        lane = lax.iota(jnp.int32, _LANES)

        def scan_half(base):
            def scan_body(j, carry):
                ca, cb = carry
                off = base + j * (2 * _LANES)
                va = ids_v[pl.ds(off, _LANES)]
                vb = ids_v[pl.ds(off + _LANES, _LANES)]
                ca = jnp.minimum(ca, jnp.where(va == 0, lane + off, _SEQ))
                cb = jnp.minimum(
                    cb, jnp.where(vb == 0, lane + (off + _LANES), _SEQ)
                )
                return ca, cb

            return scan_body

        init = jnp.full((_LANES,), _SEQ, jnp.int32)
        c0.wait()
        carry = lax.fori_loop(
            0, _NVREG_H // 2, scan_half(0), (init, init), unroll=4,
        )
        c1.wait()
        carry = lax.fori_loop(
            0, _NVREG_H // 2, scan_half(_HSEQ), carry, unroll=4,
        )
        mvec = jnp.minimum(carry[0], carry[1])
        for sh in (1, 2, 4, 8):
            mvec = jnp.minimum(
                mvec, mvec.at[lane ^ sh].get(mode="promise_in_bounds")
            )
        p = mvec[0]
        idx = (p + (_SEQ - 1)) % _SEQ
        quarter = _HID // 4
        gathers = []
        for i, sem in enumerate((s2, s3, s4, s5)):
            g = pltpu.make_async_copy(
                hs_hbm.at[b, idx, pl.ds(i * quarter, quarter)],
                row_v.at[pl.ds(i * quarter, quarter)],
                sem,
            )
            g.start()
            gathers.append(g)
        outs = []
        for i, (g, sem) in enumerate(zip(gathers, (s0, s1, s6, s7))):
            g.wait()
            o = pltpu.make_async_copy(
                row_v.at[pl.ds(i * quarter, quarter)],
                out_hbm.at[b, pl.ds(i * quarter, quarter)],
                sem,
            )
            o.start()
            outs.append(o)
        for o in outs:
            o.wait()


def kernel(input_ids, hidden_states):
    mesh = plsc.VectorSubcoreMesh(
        core_axis_name="c", subcore_axis_name="s", num_cores=1,
    )
    k = pl.kernel(
        _sc_body,
        out_type=jax.ShapeDtypeStruct((_BATCH, _HID), jnp.float32),
        mesh=mesh,
        scratch_types=[
            pltpu.VMEM((_SEQ,), jnp.int32),
            pltpu.VMEM((_HID,), jnp.float32),
            pltpu.SemaphoreType.DMA,
            pltpu.SemaphoreType.DMA,
            pltpu.SemaphoreType.DMA,
            pltpu.SemaphoreType.DMA,
            pltpu.SemaphoreType.DMA,
            pltpu.SemaphoreType.DMA,
            pltpu.SemaphoreType.DMA,
            pltpu.SemaphoreType.DMA,
        ],
    )
    return k(input_ids.astype(jnp.int32), hidden_states)


# speedup vs baseline: 1.0200x; 1.0200x over previous
import jax
import jax.numpy as jnp
from jax import lax
from jax.experimental import pallas as pl
from jax.experimental.pallas import tpu as pltpu
from jax.experimental.pallas import tpu_sc as plsc

_BATCH = 4
_SEQ = 4096
_HID = 4096
_LANES = 16
_HSEQ = _SEQ // 2
_HHID = _HID // 2
_NVREG_H = _HSEQ // _LANES


def _sc_body(ids_hbm, hs_hbm, out_hbm, ids_v, row_v, s0, s1, s2, s3):
    b = lax.axis_index("s")

    @pl.when(b < _BATCH)
    def _():
        c0 = pltpu.make_async_copy(
            ids_hbm.at[b, pl.ds(0, _HSEQ)], ids_v.at[pl.ds(0, _HSEQ)], s0
        )
        c0.start()
        c1 = pltpu.make_async_copy(
            ids_hbm.at[b, pl.ds(_HSEQ, _HSEQ)],
            ids_v.at[pl.ds(_HSEQ, _HSEQ)],
            s1,
        )
        c1.start()
        lane = lax.iota(jnp.int32, _LANES)

        def scan_half(base):
            def scan_body(j, carry):
                ca, cb = carry
                off = base + j * (2 * _LANES)
                va = ids_v[pl.ds(off, _LANES)]
                vb = ids_v[pl.ds(off + _LANES, _LANES)]
                ca = jnp.minimum(ca, jnp.where(va == 0, lane + off, _SEQ))
                cb = jnp.minimum(
                    cb, jnp.where(vb == 0, lane + (off + _LANES), _SEQ)
                )
                return ca, cb

            return scan_body

        init = jnp.full((_LANES,), _SEQ, jnp.int32)
        c0.wait()
        carry = lax.fori_loop(
            0, _NVREG_H // 2, scan_half(0), (init, init), unroll=4,
        )
        c1.wait()
        carry = lax.fori_loop(
            0, _NVREG_H // 2, scan_half(_HSEQ), carry, unroll=4,
        )
        mvec = jnp.minimum(carry[0], carry[1])
        for sh in (1, 2, 4, 8):
            mvec = jnp.minimum(
                mvec, mvec.at[lane ^ sh].get(mode="promise_in_bounds")
            )
        p = mvec[0]
        idx = (p + (_SEQ - 1)) % _SEQ
        g0 = pltpu.make_async_copy(
            hs_hbm.at[b, idx, pl.ds(0, _HHID)], row_v.at[pl.ds(0, _HHID)], s2
        )
        g0.start()
        g1 = pltpu.make_async_copy(
            hs_hbm.at[b, idx, pl.ds(_HHID, _HHID)],
            row_v.at[pl.ds(_HHID, _HHID)],
            s3,
        )
        g1.start()
        g0.wait()
        o0 = pltpu.make_async_copy(
            row_v.at[pl.ds(0, _HHID)], out_hbm.at[b, pl.ds(0, _HHID)], s0
        )
        o0.start()
        g1.wait()
        o1 = pltpu.make_async_copy(
            row_v.at[pl.ds(_HHID, _HHID)],
            out_hbm.at[b, pl.ds(_HHID, _HHID)],
            s1,
        )
        o1.start()
        o0.wait()
        o1.wait()


def kernel(input_ids, hidden_states):
    mesh = plsc.VectorSubcoreMesh(
        core_axis_name="c", subcore_axis_name="s", num_cores=1,
    )
    k = pl.kernel(
        _sc_body,
        out_type=jax.ShapeDtypeStruct((_BATCH, _HID), jnp.float32),
        mesh=mesh,
        scratch_types=[
            pltpu.VMEM((_SEQ,), jnp.int32),
            pltpu.VMEM((_HID,), jnp.float32),
            pltpu.SemaphoreType.DMA,
            pltpu.SemaphoreType.DMA,
            pltpu.SemaphoreType.DMA,
            pltpu.SemaphoreType.DMA,
        ],
    )
    return k(input_ids.astype(jnp.int32), hidden_states)


# speedup vs baseline: 1.0245x; 1.0044x over previous
import jax
import jax.numpy as jnp
from jax import lax
from jax.experimental import pallas as pl
from jax.experimental.pallas import tpu as pltpu
from jax.experimental.pallas import tpu_sc as plsc

_BATCH = 4
_SEQ = 4096
_HID = 4096
_LANES = 16
_HSEQ = _SEQ // 2
_HHID = _HID // 2
_NVREG_H = _HSEQ // _LANES


def _sc_body(ids_hbm, hs_hbm, out_hbm, ids_v, row_v, s0, s1, s2, s3):
    b = lax.axis_index("s")

    @pl.when(b < _BATCH)
    def _():
        c0 = pltpu.make_async_copy(
            ids_hbm.at[b, pl.ds(0, _HSEQ)], ids_v.at[pl.ds(0, _HSEQ)], s0
        )
        c0.start()
        c1 = pltpu.make_async_copy(
            ids_hbm.at[b, pl.ds(_HSEQ, _HSEQ)],
            ids_v.at[pl.ds(_HSEQ, _HSEQ)],
            s1,
        )
        c1.start()
        lane = lax.iota(jnp.int32, _LANES)

        def scan_half(base):
            def scan_body(j, carry):
                ca, cb = carry
                off = base + j * (2 * _LANES)
                va = ids_v[pl.ds(off, _LANES)]
                vb = ids_v[pl.ds(off + _LANES, _LANES)]
                ca = jnp.minimum(ca, jnp.where(va == 0, lane + off, _SEQ))
                cb = jnp.minimum(
                    cb, jnp.where(vb == 0, lane + (off + _LANES), _SEQ)
                )
                return ca, cb

            return scan_body

        init = jnp.full((_LANES,), _SEQ, jnp.int32)
        c0.wait()
        carry = lax.fori_loop(
            0, _NVREG_H // 2, scan_half(0), (init, init), unroll=4,
        )
        c1.wait()
        carry = lax.fori_loop(
            0, _NVREG_H // 2, scan_half(_HSEQ), carry, unroll=4,
        )
        mvec = jnp.minimum(carry[0], carry[1])
        for sh in (1, 2, 4, 8):
            mvec = jnp.minimum(
                mvec, mvec.at[lane ^ sh].get(mode="promise_in_bounds")
            )
        p = mvec[0]
        idx = (p + (_SEQ - 1)) % _SEQ
        g0 = pltpu.make_async_copy(
            hs_hbm.at[b, idx, pl.ds(0, _HHID)], row_v.at[pl.ds(0, _HHID)], s2
        )
        g0.start()
        g1 = pltpu.make_async_copy(
            hs_hbm.at[b, idx, pl.ds(_HHID, _HHID)],
            row_v.at[pl.ds(_HHID, _HHID)],
            s3,
        )
        g1.start()
        g0.wait()
        o0 = pltpu.make_async_copy(
            row_v.at[pl.ds(0, _HHID)], out_hbm.at[b, pl.ds(0, _HHID)], s0
        )
        o0.start()
        g1.wait()
        o1 = pltpu.make_async_copy(
            row_v.at[pl.ds(_HHID, _HHID)],
            out_hbm.at[b, pl.ds(_HHID, _HHID)],
            s1,
        )
        o1.start()
        o0.wait()
        o1.wait()


def kernel(input_ids, hidden_states):
    mesh = plsc.VectorSubcoreMesh(
        core_axis_name="c", subcore_axis_name="s", num_cores=1,
    )
    k = pl.kernel(
        _sc_body,
        out_type=jax.ShapeDtypeStruct((_BATCH, _HID), jnp.float32),
        mesh=mesh,
        compiler_params=pltpu.CompilerParams(
            skip_device_barrier=True,
            disable_semaphore_checks=True,
            disable_bounds_checks=True,
        ),
        scratch_types=[
            pltpu.VMEM((_SEQ,), jnp.int32),
            pltpu.VMEM((_HID,), jnp.float32),
            pltpu.SemaphoreType.DMA,
            pltpu.SemaphoreType.DMA,
            pltpu.SemaphoreType.DMA,
            pltpu.SemaphoreType.DMA,
        ],
    )
    return k(input_ids.astype(jnp.int32), hidden_states)


# speedup vs baseline: 1.0482x; 1.0231x over previous
import jax
import jax.numpy as jnp
from jax import lax
from jax.experimental import pallas as pl
from jax.experimental.pallas import tpu as pltpu
from jax.experimental.pallas import tpu_sc as plsc

_BATCH = 4
_SEQ = 4096
_HID = 4096
_LANES = 16
_HSEQ = _SEQ // 2
_HHID = _HID // 2
_NVREG_H = _HSEQ // _LANES


def _sc_body(ids_hbm, hs_hbm, out_hbm, ids_v, row_v, s0, s1, s2, s3):
    b = lax.axis_index("s")

    @pl.when(b < _BATCH)
    def _():
        c0 = pltpu.make_async_copy(
            ids_hbm.at[b, pl.ds(0, _HSEQ)], ids_v.at[pl.ds(0, _HSEQ)], s0
        )
        c0.start()
        c1 = pltpu.make_async_copy(
            ids_hbm.at[b, pl.ds(_HSEQ, _HSEQ)],
            ids_v.at[pl.ds(_HSEQ, _HSEQ)],
            s1,
        )
        c1.start()
        g0 = pltpu.make_async_copy(
            hs_hbm.at[b, _SEQ - 1, pl.ds(0, _HHID)],
            row_v.at[pl.ds(0, _HHID)],
            s2,
        )
        g0.start()
        g1 = pltpu.make_async_copy(
            hs_hbm.at[b, _SEQ - 1, pl.ds(_HHID, _HHID)],
            row_v.at[pl.ds(_HHID, _HHID)],
            s3,
        )
        g1.start()
        lane = lax.iota(jnp.int32, _LANES)

        def scan_half(base):
            def scan_body(j, carry):
                ca, cb = carry
                off = base + j * (2 * _LANES)
                va = ids_v[pl.ds(off, _LANES)]
                vb = ids_v[pl.ds(off + _LANES, _LANES)]
                ca = jnp.minimum(ca, jnp.where(va == 0, lane + off, _SEQ))
                cb = jnp.minimum(
                    cb, jnp.where(vb == 0, lane + (off + _LANES), _SEQ)
                )
                return ca, cb

            return scan_body

        init = jnp.full((_LANES,), _SEQ, jnp.int32)
        c0.wait()
        carry = lax.fori_loop(
            0, _NVREG_H // 2, scan_half(0), (init, init), unroll=4,
        )
        c1.wait()
        carry = lax.fori_loop(
            0, _NVREG_H // 2, scan_half(_HSEQ), carry, unroll=4,
        )
        mvec = jnp.minimum(carry[0], carry[1])
        for sh in (1, 2, 4, 8):
            mvec = jnp.minimum(
                mvec, mvec.at[lane ^ sh].get(mode="promise_in_bounds")
            )
        p = mvec[0]
        idx = (p + (_SEQ - 1)) % _SEQ
        g0.wait()
        g1.wait()

        @pl.when(idx != _SEQ - 1)
        def _():
            pltpu.sync_copy(hs_hbm.at[b, idx], row_v)

        o0 = pltpu.make_async_copy(
            row_v.at[pl.ds(0, _HHID)], out_hbm.at[b, pl.ds(0, _HHID)], s0
        )
        o0.start()
        o1 = pltpu.make_async_copy(
            row_v.at[pl.ds(_HHID, _HHID)],
            out_hbm.at[b, pl.ds(_HHID, _HHID)],
            s1,
        )
        o1.start()
        o0.wait()
        o1.wait()


def kernel(input_ids, hidden_states):
    mesh = plsc.VectorSubcoreMesh(
        core_axis_name="c", subcore_axis_name="s", num_cores=1,
    )
    k = pl.kernel(
        _sc_body,
        out_type=jax.ShapeDtypeStruct((_BATCH, _HID), jnp.float32),
        mesh=mesh,
        scratch_types=[
            pltpu.VMEM((_SEQ,), jnp.int32),
            pltpu.VMEM((_HID,), jnp.float32),
            pltpu.SemaphoreType.DMA,
            pltpu.SemaphoreType.DMA,
            pltpu.SemaphoreType.DMA,
            pltpu.SemaphoreType.DMA,
        ],
    )
    return k(input_ids.astype(jnp.int32), hidden_states)
